# baseline (speedup 1.0000x reference)
import functools

import jax
import jax.numpy as jnp
from jax import lax
from jax.experimental import pallas as pl
from jax.experimental.pallas import tpu as pltpu
from jax.experimental.pallas import tpu_sc as plsc

_N = 10000
_E = 320000
_D = 128
_NC = 2
_NS = 16
_BW = 80
_NB = _E // (_NC * _NS * _BW)
_NP = 10016
_RPT = _NP // _NS


def _seg_body(h_hbm, hbf_hbm, src_hbm, dst_hbm, out_hbm, sidx, didx,
              bf_a, bf_b, rf_a, rf_b, acc, gsem_a, gsem_b, ssem_a, ssem_b):
    c = lax.axis_index("c")
    s = lax.axis_index("s")
    wid = s * _NC + c
    r0 = s * _RPT
    cb = wid * _NB
    pltpu.sync_copy(src_hbm.at[pl.ds(cb, _NB)], sidx)
    pltpu.sync_copy(dst_hbm.at[pl.ds(cb, _NB)], didx)
    pltpu.sync_copy(h_hbm.at[pl.ds(r0, _RPT)], acc.at[pl.ds(r0, _RPT)])
    plsc.subcore_barrier()

    def start_g(j, bf, gsem):
        pltpu.async_copy(hbf_hbm.at[sidx.at[j]], bf, gsem)

    def wait_g(j, bf, gsem):
        pltpu.make_async_copy(hbf_hbm.at[sidx.at[j]], bf, gsem).wait()

    def start_s(j, rf, ssem):
        pltpu.async_copy(rf, acc.at[didx.at[j]], ssem, add=True)

    def wait_s(j, rf, ssem):
        pltpu.make_async_copy(rf, acc.at[didx.at[j]], ssem).wait()

    def convert(bf, rf):
        def crow(r, carry):
            for q in range(4):
                v = plsc.bitcast(bf[r, pl.ds(32 * q, 32)], jnp.int32)
                lo = plsc.bitcast(v << 16, jnp.float32)
                hi = plsc.bitcast(v & jnp.int32(-65536), jnp.float32)
                rf[r, pl.ds(16 * q, 16)] = lo
                rf[r, pl.ds(64 + 16 * q, 16)] = hi
            return carry
        lax.fori_loop(0, _BW, crow, 0)

    start_g(0, bf_a, gsem_a)
    start_g(1, bf_b, gsem_b)
    wait_g(0, bf_a, gsem_a)
    convert(bf_a, rf_a)
    start_g(2, bf_a, gsem_a)
    start_s(0, rf_a, ssem_a)
    wait_g(1, bf_b, gsem_b)
    convert(bf_b, rf_b)
    start_g(3, bf_b, gsem_b)
    start_s(1, rf_b, ssem_b)

    def body(i, carry):
        j = 2 * i + 2
        wait_g(j, bf_a, gsem_a)
        wait_s(j - 2, rf_a, ssem_a)
        convert(bf_a, rf_a)
        start_g(j + 2, bf_a, gsem_a)
        start_s(j, rf_a, ssem_a)
        wait_g(j + 1, bf_b, gsem_b)
        wait_s(j - 1, rf_b, ssem_b)
        convert(bf_b, rf_b)
        start_g(j + 3, bf_b, gsem_b)
        start_s(j + 1, rf_b, ssem_b)
        return carry

    lax.fori_loop(0, (_NB - 5) // 2, body, 0)
    wait_g(_NB - 3, bf_a, gsem_a)
    wait_s(_NB - 5, rf_a, ssem_a)
    convert(bf_a, rf_a)
    start_g(_NB - 1, bf_a, gsem_a)
    start_s(_NB - 3, rf_a, ssem_a)
    wait_g(_NB - 2, bf_b, gsem_b)
    wait_s(_NB - 4, rf_b, ssem_b)
    convert(bf_b, rf_b)
    start_s(_NB - 2, rf_b, ssem_b)
    wait_g(_NB - 1, bf_a, gsem_a)
    wait_s(_NB - 3, rf_a, ssem_a)
    convert(bf_a, rf_a)
    start_s(_NB - 1, rf_a, ssem_a)
    wait_s(_NB - 2, rf_b, ssem_b)
    wait_s(_NB - 1, rf_a, ssem_a)
    plsc.subcore_barrier()
    pltpu.sync_copy(acc.at[pl.ds(r0, _RPT)], out_hbm.at[c, pl.ds(r0, _RPT)])


@jax.jit
def _seg_call(h, hbf, src2d, dst2d):
    mesh = plsc.VectorSubcoreMesh(core_axis_name="c", subcore_axis_name="s")
    return pl.kernel(
        _seg_body,
        out_type=jax.ShapeDtypeStruct((_NC, _NP, _D), jnp.float32),
        mesh=mesh,
        scratch_types=[
            pltpu.VMEM((_NB, _BW), jnp.int32),
            pltpu.VMEM((_NB, _BW), jnp.int32),
            pltpu.VMEM((_BW, _D), jnp.bfloat16),
            pltpu.VMEM((_BW, _D), jnp.bfloat16),
            pltpu.VMEM((_BW, _D), jnp.float32),
            pltpu.VMEM((_BW, _D), jnp.float32),
            pltpu.VMEM_SHARED((_NP, _D), jnp.float32),
            pltpu.SemaphoreType.DMA,
            pltpu.SemaphoreType.DMA,
            pltpu.SemaphoreType.DMA,
            pltpu.SemaphoreType.DMA,
        ],
        compiler_params=pltpu.CompilerParams(use_tc_tiling_on_sc=False, needs_layout_passes=False),
    )(h, hbf, src2d, dst2d)


def _bn_masked(t, rm, g, be):
    m = jnp.sum(t * rm, axis=0, keepdims=True) * (1.0 / _N)
    d = (t - m) * rm
    v = jnp.sum(d * d, axis=0, keepdims=True) * (1.0 / _N)
    return d * lax.rsqrt(v + 1e-5) * g + be


def _mlp_body(final_bn, h_ref, p_ref, w1_ref, b1_ref, g1_ref, be1_ref,
              w2_ref, b2_ref, g2_ref, be2_ref, o_ref):
    rows = lax.broadcasted_iota(jnp.int32, (_NP, 1), 0)
    rm = jnp.where(rows < _N, 1.0, 0.0)
    rst = p_ref[0] + p_ref[1] - h_ref[...]
    t = jnp.dot(rst, w1_ref[...], preferred_element_type=jnp.float32) + b1_ref[...]
    t = jnp.maximum(_bn_masked(t, rm, g1_ref[...], be1_ref[...]), 0.0)
    t = jnp.dot(t, w2_ref[...], preferred_element_type=jnp.float32) + b2_ref[...]
    if final_bn:
        t = jnp.maximum(_bn_masked(t, rm, g2_ref[...], be2_ref[...]), 0.0)
        o_ref[...] = t * rm
    else:
        o_ref[...] = t[:_N]


def _mlp_call(h, p, w1, b1, g1, be1, w2, b2, g2, be2, final_bn):
    vecs = [vv.reshape(1, _D) for vv in (b1, g1, be1, b2, g2, be2)]
    out_rows = _NP if final_bn else _N
    return pl.pallas_call(
        functools.partial(_mlp_body, final_bn),
        out_shape=jax.ShapeDtypeStruct((out_rows, _D), jnp.float32),
    )(h, p, w1, vecs[0], vecs[1], vecs[2], w2, vecs[3], vecs[4],
      vecs[5])


def kernel(x, edge_index, l0_w1, l0_b1, l0_g1, l0_be1, l0_w2, l0_b2, l0_g2,
           l0_be2, l1_w1, l1_b1, l1_g1, l1_be1, l1_w2, l1_b2):
    src2d = edge_index[0].reshape(_E // _BW, _BW)
    dst2d = edge_index[1].reshape(_E // _BW, _BW)
    colmap = jnp.stack(
        [jnp.arange(64, dtype=jnp.int32),
         64 + jnp.arange(64, dtype=jnp.int32)], axis=1).reshape(128)
    xp = jnp.pad(x, ((0, _NP - _N), (0, 0)))
    xbf = xp[:, colmap].astype(jnp.bfloat16)
    p0 = _seg_call(xp, xbf, src2d, dst2d)
    h1 = _mlp_call(xp, p0, l0_w1, l0_b1, l0_g1, l0_be1, l0_w2, l0_b2,
                   l0_g2, l0_be2, True)
    h1bf = h1[:, colmap].astype(jnp.bfloat16)
    p1 = _seg_call(h1, h1bf, src2d, dst2d)
    out = _mlp_call(h1, p1, l1_w1, l1_b1, l1_g1, l1_be1, l1_w2, l1_b2,
                    l1_b2, l1_b2, False)
    return out


# speedup vs baseline: 1.9482x; 1.9482x over previous
import functools

import jax
import jax.numpy as jnp
from jax import lax
from jax.experimental import pallas as pl
from jax.experimental.pallas import tpu as pltpu
from jax.experimental.pallas import tpu_sc as plsc

_N = 10000
_E = 320000
_D = 128
_NC = 2
_NS = 16
_BW = 80
_NB = _E // (_NC * _NS * _BW)
_NP = 10016
_RPT = _NP // _NS


def _seg_body(h_hbm, src_hbm, dst_hbm, out_hbm, sidx, didx, rows_0, rows_1,
              rows_2, acc, gsem_0, gsem_1, gsem_2, ssem_0, ssem_1, ssem_2):
    c = lax.axis_index("c")
    s = lax.axis_index("s")
    wid = s * _NC + c
    r0 = s * _RPT
    cb = wid * _NB
    pltpu.sync_copy(src_hbm.at[pl.ds(cb, _NB)], sidx)
    pltpu.sync_copy(dst_hbm.at[pl.ds(cb, _NB)], didx)
    pltpu.sync_copy(h_hbm.at[pl.ds(r0, _RPT)], acc.at[pl.ds(r0, _RPT)])
    plsc.subcore_barrier()

    def start_g(j, rows, gsem):
        pltpu.async_copy(h_hbm.at[sidx.at[j]], rows, gsem)

    def wait_g(j, rows, gsem):
        pltpu.make_async_copy(h_hbm.at[sidx.at[j]], rows, gsem).wait()

    def start_s(j, rows, ssem):
        pltpu.async_copy(rows, acc.at[didx.at[j]], ssem, add=True)

    def wait_s(j, rows, ssem):
        pltpu.make_async_copy(rows, acc.at[didx.at[j]], ssem).wait()

    bufs = (rows_0, rows_1, rows_2)
    gsems = (gsem_0, gsem_1, gsem_2)
    ssems = (ssem_0, ssem_1, ssem_2)

    def step(j, b, has_swait=True, prefetch=True):
        bp = (b - 1) % 3
        wait_g(j, bufs[b], gsems[b])
        start_s(j, bufs[b], ssems[b])
        if has_swait:
            wait_s(j - 1, bufs[bp], ssems[bp])
        if prefetch:
            start_g(j + 2, bufs[bp], gsems[bp])

    start_g(0, rows_0, gsem_0)
    start_g(1, rows_1, gsem_1)
    wait_g(0, rows_0, gsem_0)
    start_s(0, rows_0, ssem_0)
    start_g(2, rows_2, gsem_2)
    step(1, 1)

    def body(i, carry):
        j = 3 * i + 2
        step(j, 2)
        step(j + 1, 0)
        step(j + 2, 1)
        return carry

    lax.fori_loop(0, (_NB - 5) // 3, body, 0)
    step(_NB - 3, (_NB - 3) % 3)
    step(_NB - 2, (_NB - 2) % 3, prefetch=False)
    step(_NB - 1, (_NB - 1) % 3, prefetch=False)
    wait_s(_NB - 1, bufs[(_NB - 1) % 3], ssems[(_NB - 1) % 3])
    plsc.subcore_barrier()
    pltpu.sync_copy(acc.at[pl.ds(r0, _RPT)], out_hbm.at[c, pl.ds(r0, _RPT)])


@jax.jit
def _seg_call(h, src2d, dst2d):
    mesh = plsc.VectorSubcoreMesh(core_axis_name="c", subcore_axis_name="s")
    return pl.kernel(
        _seg_body,
        out_type=jax.ShapeDtypeStruct((_NC, _NP, _D), jnp.float32),
        mesh=mesh,
        scratch_types=[
            pltpu.VMEM((_NB, _BW), jnp.int32),
            pltpu.VMEM((_NB, _BW), jnp.int32),
            pltpu.VMEM((_BW, _D), jnp.float32),
            pltpu.VMEM((_BW, _D), jnp.float32),
            pltpu.VMEM((_BW, _D), jnp.float32),
            pltpu.VMEM_SHARED((_NP, _D), jnp.float32),
            pltpu.SemaphoreType.DMA,
            pltpu.SemaphoreType.DMA,
            pltpu.SemaphoreType.DMA,
            pltpu.SemaphoreType.DMA,
            pltpu.SemaphoreType.DMA,
            pltpu.SemaphoreType.DMA,
        ],
        compiler_params=pltpu.CompilerParams(use_tc_tiling_on_sc=False),
    )(h, src2d, dst2d)


def _bn_masked(t, rm, g, be):
    m = jnp.sum(t * rm, axis=0, keepdims=True) * (1.0 / _N)
    d = (t - m) * rm
    v = jnp.sum(d * d, axis=0, keepdims=True) * (1.0 / _N)
    return d * lax.rsqrt(v + 1e-5) * g + be


def _mlp_body(final_bn, h_ref, p_ref, w1_ref, b1_ref, g1_ref, be1_ref,
              w2_ref, b2_ref, g2_ref, be2_ref, o_ref):
    rows = lax.broadcasted_iota(jnp.int32, (_NP, 1), 0)
    rm = jnp.where(rows < _N, 1.0, 0.0)
    rst = p_ref[0] + p_ref[1] - h_ref[...]
    t = jnp.dot(rst, w1_ref[...], preferred_element_type=jnp.float32) + b1_ref[...]
    t = jnp.maximum(_bn_masked(t, rm, g1_ref[...], be1_ref[...]), 0.0)
    t = jnp.dot(t, w2_ref[...], preferred_element_type=jnp.float32) + b2_ref[...]
    if final_bn:
        t = jnp.maximum(_bn_masked(t, rm, g2_ref[...], be2_ref[...]), 0.0)
        o_ref[...] = t * rm
    else:
        o_ref[...] = t[:_N]


def _mlp_call(h, p, w1, b1, g1, be1, w2, b2, g2, be2, final_bn):
    vecs = [vv.reshape(1, _D) for vv in (b1, g1, be1, b2, g2, be2)]
    out_rows = _NP if final_bn else _N
    return pl.pallas_call(
        functools.partial(_mlp_body, final_bn),
        out_shape=jax.ShapeDtypeStruct((out_rows, _D), jnp.float32),
    )(h, p, w1, vecs[0], vecs[1], vecs[2], w2, vecs[3], vecs[4],
      vecs[5])


def kernel(x, edge_index, l0_w1, l0_b1, l0_g1, l0_be1, l0_w2, l0_b2, l0_g2,
           l0_be2, l1_w1, l1_b1, l1_g1, l1_be1, l1_w2, l1_b2):
    src2d = edge_index[0].reshape(_E // _BW, _BW)
    dst2d = edge_index[1].reshape(_E // _BW, _BW)
    xp = jnp.pad(x, ((0, _NP - _N), (0, 0)))
    p0 = _seg_call(xp, src2d, dst2d)
    h1 = _mlp_call(xp, p0, l0_w1, l0_b1, l0_g1, l0_be1, l0_w2, l0_b2,
                   l0_g2, l0_be2, True)
    p1 = _seg_call(h1, src2d, dst2d)
    out = _mlp_call(h1, p1, l1_w1, l1_b1, l1_g1, l1_be1, l1_w2, l1_b2,
                    l1_b2, l1_b2, False)
    return out
